# baseline (speedup 1.0000x reference)
import jax
import jax.numpy as jnp
from jax import lax
from jax.experimental import pallas as pl
from jax.experimental.pallas import tpu as pltpu
from jax.experimental.pallas import tpu_sc as plsc

B = 1024
S = 200
D = 128
NV = 65536
NC = 2
NS = 16
NW = NC * NS
B_PER_W = B // NW
SP = 208
NG = (S + 15) // 16
XL = S * 16

_W = [float(2.0 ** (15 - j)) for j in range(16)]


def _pairwise_dot(cols):
    def bf16_round(c):
        u = lax.bitcast_convert_type(c, jnp.int32)
        lsb = jnp.bitwise_and(lax.shift_right_logical(u, 16), 1)
        u = jnp.bitwise_and(u + 0x7FFF + lsb, jnp.int32(-65536))
        return lax.bitcast_convert_type(u, jnp.float32)

    terms = [bf16_round(c) * _W[j] for j, c in enumerate(cols)]
    while len(terms) > 1:
        terms = [terms[i] + terms[i + 1] for i in range(0, len(terms), 2)]
    return terms[0]


def _body(x_h, val_h, enc_h, pos_h, cls_h, out_h,
          x_f, idx_v, vals_v, enc_v, pos_v, cls_v,
          in_sem, gat_sem, out_sem):
    wid = lax.axis_index("s") * NC + lax.axis_index("c")
    iota = lax.iota(jnp.int32, 16)
    b0 = wid * B_PER_W

    pltpu.sync_copy(cls_h.at[0], cls_v)

    def slot2(bl):
        return jnp.bitwise_and(bl, 1)

    def slot3(bl):
        return lax.rem(bl, 3)

    def in_copies(bl):
        q = slot3(bl)
        b = b0 + bl
        return (
            pltpu.make_async_copy(x_h.at[b], x_f.at[pl.ds(q * XL, XL)],
                                  in_sem.at[q]),
            pltpu.make_async_copy(val_h.at[pl.ds(b * S, S)],
                                  vals_v.at[pl.ds(q * SP, S)], in_sem.at[q]),
        )

    def enc_copies(bl):
        p = slot2(bl)
        q = slot3(bl)
        h = SP // 2
        return tuple(
            pltpu.make_async_copy(
                enc_h.at[idx_v.at[pl.ds(q * SP + off, h)]],
                enc_v.at[p, pl.ds(off, h)], gat_sem.at[p])
            for off in (0, h))

    def pos_copies(bl):
        p = slot2(bl)
        q = slot3(bl)
        h = SP // 2
        return tuple(
            pltpu.make_async_copy(
                pos_h.at[vals_v.at[pl.ds(q * SP + off, h)]],
                pos_v.at[p, pl.ds(off, h)], gat_sem.at[p])
            for off in (0, h))

    def gather_copies(bl):
        return enc_copies(bl) + pos_copies(bl)

    def out_copy(bl):
        p = slot2(bl)
        return pltpu.make_async_copy(
            pos_v.at[p, pl.ds(0, S + 1)], out_h.at[b0 + bl], out_sem.at[p])

    def idx_stage(bl):
        q = slot3(bl)

        @plsc.parallel_loop(0, NG)
        def _(g):
            base = jnp.minimum(g * 16, S - 16)
            flat = (base + iota) * 16
            cols = [plsc.load_gather(x_f, [q * XL + flat + j])
                    for j in range(16)]
            y = _pairwise_dot(cols)
            idx_v[pl.ds(q * SP + base, 16)] = jnp.clip(
                y.astype(jnp.int32), 0, NV - 1)

        tail = idx_v[pl.ds(q * SP + SP - 16, 16)]
        pad = wid * 16 + iota
        idx_v[pl.ds(q * SP + SP - 16, 16)] = jnp.where(
            iota < 16 - (SP - S), tail, pad)

        def mx_grp(g, m):
            base = jnp.minimum(g * 16, S - 16)
            return jnp.maximum(m, vals_v[pl.ds(q * SP + base, 16)])
        m = lax.fori_loop(0, NG, mx_grp, jnp.zeros((16,), jnp.int32),
                          unroll=True)
        vmax1 = jnp.max(m) + 1
        tail = vals_v[pl.ds(q * SP + SP - 16, 16)]
        vals_v[pl.ds(q * SP + SP - 16, 16)] = jnp.where(
            iota < 16 - (SP - S), tail, jnp.full((16,), 0, jnp.int32) + vmax1)

    def main_stage(bl):
        p = slot2(bl)
        for k in range(8):
            sl = pl.ds(k * 16, 16)
            pos_v[p, 0, sl] = pos_v[p, 0, sl] + cls_v[sl]

        @plsc.parallel_loop(0, S, unroll=1)
        def _(si):
            acc = jnp.zeros((16,), jnp.float32)
            es = []
            for k in range(8):
                e = enc_v[p, si, pl.ds(k * 16, 16)]
                es.append(e)
                acc = acc + e * e
            tot = jnp.full((16,), jnp.sum(acc))
            bits = lax.shift_right_logical(
                lax.bitcast_convert_type(tot, jnp.int32), 1)
            r = lax.bitcast_convert_type(jnp.int32(0x5F3759DF) - bits,
                                         jnp.float32)
            for _ in range(3):
                r = r * (1.5 - 0.5 * tot * r * r)
            inv = 1.0 / jnp.maximum(tot * r, 1e-12)
            for k in range(8):
                sl = pl.ds(k * 16, 16)
                pos_v[p, si + 1, sl] = pos_v[p, si + 1, sl] + es[k] * inv

    for c in in_copies(0):
        c.start()
    for c in in_copies(0):
        c.wait()
    idx_stage(0)
    for c in gather_copies(0):
        c.start()
    for c in in_copies(1):
        c.start()
    for c in in_copies(1):
        c.wait()
    idx_stage(1)

    def pipe(bl, _):
        @pl.when(bl + 2 < B_PER_W)
        def _():
            for c in in_copies(bl + 2):
                c.start()

        for c in enc_copies(bl + 1):
            c.start()

        @pl.when(bl >= 1)
        def _():
            out_copy(bl - 1).wait()

        for c in pos_copies(bl + 1):
            c.start()
        for c in gather_copies(bl):
            c.wait()
        main_stage(bl)
        out_copy(bl).start()

        @pl.when(bl + 2 < B_PER_W)
        def _():
            for c in in_copies(bl + 2):
                c.wait()
            idx_stage(bl + 2)
        return 0

    lax.fori_loop(0, B_PER_W - 1, pipe, 0)
    bl_last = B_PER_W - 1
    for c in gather_copies(bl_last):
        c.wait()
    main_stage(bl_last)
    out_copy(bl_last).start()
    out_copy(bl_last - 1).wait()
    out_copy(bl_last).wait()


def kernel(x, values, enc_table, pos_table, cls_token, bit_weights):
    del bit_weights
    mesh = plsc.VectorSubcoreMesh(
        core_axis_name="c", subcore_axis_name="s",
        num_cores=NC, num_subcores=NS)
    f = pl.kernel(
        _body,
        out_type=jax.ShapeDtypeStruct((B, S + 1, D), jnp.float32),
        mesh=mesh,
        compiler_params=pltpu.CompilerParams(needs_layout_passes=False),
        scratch_types=[
            pltpu.VMEM((3 * XL,), jnp.float32),
            pltpu.VMEM((3 * SP,), jnp.int32),
            pltpu.VMEM((3 * SP,), jnp.int32),
            pltpu.VMEM((2, SP, D), jnp.float32),
            pltpu.VMEM((2, SP, D), jnp.float32),
            pltpu.VMEM((D,), jnp.float32),
            pltpu.SemaphoreType.DMA((3,)),
            pltpu.SemaphoreType.DMA((2,)),
            pltpu.SemaphoreType.DMA((2,)),
        ],
    )
    return f(x.reshape(B, XL), values.reshape(B * S), enc_table,
             pos_table, cls_token)


# speedup vs baseline: 1.0240x; 1.0240x over previous
import jax
import jax.numpy as jnp
from jax import lax
from jax.experimental import pallas as pl
from jax.experimental.pallas import tpu as pltpu
from jax.experimental.pallas import tpu_sc as plsc

B = 1024
S = 200
D = 128
NV = 65536
NC = 2
NS = 16
NW = NC * NS
B_PER_W = B // NW
SP = 208
NG = (S + 15) // 16
XL = S * 16

_W = [float(2.0 ** (15 - j)) for j in range(16)]


def _pairwise_dot(cols):
    def bf16_round(c):
        u = lax.bitcast_convert_type(c, jnp.int32)
        lsb = jnp.bitwise_and(lax.shift_right_logical(u, 16), 1)
        u = jnp.bitwise_and(u + 0x7FFF + lsb, jnp.int32(-65536))
        return lax.bitcast_convert_type(u, jnp.float32)

    terms = [bf16_round(c) * _W[j] for j, c in enumerate(cols)]
    while len(terms) > 1:
        terms = [terms[i] + terms[i + 1] for i in range(0, len(terms), 2)]
    return terms[0]


def _body(x_h, val_h, enc_h, pos_h, cls_h, out_h,
          x_f, idx_v, vals_v, enc_v, pos_v, cls_v,
          in_sem, gat_sem, out_sem):
    wid = lax.axis_index("s") * NC + lax.axis_index("c")
    iota = lax.iota(jnp.int32, 16)
    b0 = wid * B_PER_W

    pltpu.sync_copy(cls_h.at[0], cls_v)

    def slot2(bl):
        return jnp.bitwise_and(bl, 1)

    def slot3(bl):
        return lax.rem(bl, 3)

    def in_copies(bl):
        q = slot3(bl)
        b = b0 + bl
        return (
            pltpu.make_async_copy(x_h.at[b], x_f.at[pl.ds(q * XL, XL)],
                                  in_sem.at[q]),
            pltpu.make_async_copy(val_h.at[pl.ds(b * S, S)],
                                  vals_v.at[pl.ds(q * SP, S)], in_sem.at[q]),
        )

    def enc_copies(bl):
        p = slot2(bl)
        q = slot3(bl)
        h = SP // 2
        return tuple(
            pltpu.make_async_copy(
                enc_h.at[idx_v.at[pl.ds(q * SP + off, h)]],
                enc_v.at[p, pl.ds(off, h)], gat_sem.at[p])
            for off in (0, h))

    def pos_copies(bl):
        p = slot2(bl)
        q = slot3(bl)
        h = SP // 2
        return tuple(
            pltpu.make_async_copy(
                pos_h.at[vals_v.at[pl.ds(q * SP + off, h)]],
                pos_v.at[p, pl.ds(off, h)], gat_sem.at[p])
            for off in (0, h))

    def gather_copies(bl):
        return enc_copies(bl) + pos_copies(bl)

    def out_copy(bl):
        p = slot2(bl)
        return pltpu.make_async_copy(
            pos_v.at[p, pl.ds(0, S + 1)], out_h.at[b0 + bl], out_sem.at[p])

    def idx_stage(bl):
        q = slot3(bl)

        @plsc.parallel_loop(0, NG)
        def _(g):
            base = jnp.minimum(g * 16, S - 16)
            flat = (base + iota) * 16
            cols = [plsc.load_gather(x_f, [q * XL + flat + j])
                    for j in range(16)]
            y = _pairwise_dot(cols)
            idx_v[pl.ds(q * SP + base, 16)] = jnp.clip(
                y.astype(jnp.int32), 0, NV - 1)

        tail = idx_v[pl.ds(q * SP + SP - 16, 16)]
        pad = wid * 16 + iota
        idx_v[pl.ds(q * SP + SP - 16, 16)] = jnp.where(
            iota < 16 - (SP - S), tail, pad)

        def mx_grp(g, m):
            base = jnp.minimum(g * 16, S - 16)
            return jnp.maximum(m, vals_v[pl.ds(q * SP + base, 16)])
        m = lax.fori_loop(0, NG, mx_grp, jnp.zeros((16,), jnp.int32),
                          unroll=True)
        vmax1 = jnp.max(m) + 1
        tail = vals_v[pl.ds(q * SP + SP - 16, 16)]
        vals_v[pl.ds(q * SP + SP - 16, 16)] = jnp.where(
            iota < 16 - (SP - S), tail, jnp.full((16,), 0, jnp.int32) + vmax1)

    def main_stage(bl):
        p = slot2(bl)
        for k in range(8):
            sl = pl.ds(k * 16, 16)
            pos_v[p, 0, sl] = pos_v[p, 0, sl] + cls_v[sl]

        @plsc.parallel_loop(0, S, unroll=2)
        def _(si):
            acc = jnp.zeros((16,), jnp.float32)
            es = []
            for k in range(8):
                e = enc_v[p, si, pl.ds(k * 16, 16)]
                es.append(e)
                acc = acc + e * e
            tot = jnp.full((16,), jnp.sum(acc))
            bits = lax.shift_right_logical(
                lax.bitcast_convert_type(tot, jnp.int32), 1)
            r = lax.bitcast_convert_type(jnp.int32(0x5F3759DF) - bits,
                                         jnp.float32)
            for _ in range(3):
                r = r * (1.5 - 0.5 * tot * r * r)
            inv = 1.0 / jnp.maximum(tot * r, 1e-12)
            for k in range(8):
                sl = pl.ds(k * 16, 16)
                pos_v[p, si + 1, sl] = pos_v[p, si + 1, sl] + es[k] * inv

    for c in in_copies(0):
        c.start()
    for c in in_copies(0):
        c.wait()
    idx_stage(0)
    for c in gather_copies(0):
        c.start()
    for c in in_copies(1):
        c.start()
    for c in in_copies(1):
        c.wait()
    idx_stage(1)

    def pipe(bl, _):
        @pl.when(bl + 2 < B_PER_W)
        def _():
            for c in in_copies(bl + 2):
                c.start()

        for c in enc_copies(bl + 1):
            c.start()

        @pl.when(bl >= 1)
        def _():
            out_copy(bl - 1).wait()

        for c in pos_copies(bl + 1):
            c.start()
        for c in gather_copies(bl):
            c.wait()
        main_stage(bl)
        out_copy(bl).start()

        @pl.when(bl + 2 < B_PER_W)
        def _():
            for c in in_copies(bl + 2):
                c.wait()
            idx_stage(bl + 2)
        return 0

    lax.fori_loop(0, B_PER_W - 1, pipe, 0)
    bl_last = B_PER_W - 1
    for c in gather_copies(bl_last):
        c.wait()
    main_stage(bl_last)
    out_copy(bl_last).start()
    out_copy(bl_last - 1).wait()
    out_copy(bl_last).wait()


def kernel(x, values, enc_table, pos_table, cls_token, bit_weights):
    del bit_weights
    mesh = plsc.VectorSubcoreMesh(
        core_axis_name="c", subcore_axis_name="s",
        num_cores=NC, num_subcores=NS)
    f = pl.kernel(
        _body,
        out_type=jax.ShapeDtypeStruct((B, S + 1, D), jnp.float32),
        mesh=mesh,
        compiler_params=pltpu.CompilerParams(needs_layout_passes=False),
        scratch_types=[
            pltpu.VMEM((3 * XL,), jnp.float32),
            pltpu.VMEM((3 * SP,), jnp.int32),
            pltpu.VMEM((3 * SP,), jnp.int32),
            pltpu.VMEM((2, SP, D), jnp.float32),
            pltpu.VMEM((2, SP, D), jnp.float32),
            pltpu.VMEM((D,), jnp.float32),
            pltpu.SemaphoreType.DMA((3,)),
            pltpu.SemaphoreType.DMA((2,)),
            pltpu.SemaphoreType.DMA((2,)),
        ],
    )
    return f(x.reshape(B, XL), values.reshape(B * S), enc_table,
             pos_table, cls_token)
